# Initial kernel scaffold; baseline (speedup 1.0000x reference)
import jax
import jax.numpy as jnp
from jax.experimental import pallas as pl


def kernel(x, w1, s1, b1, w2, s2, b2, w3, s3, b3):
    raise NotImplementedError("write your pallas kernel here")


# speedup vs baseline: 19.4494x; 19.4494x over previous
import jax
import jax.numpy as jnp
from jax.experimental import pallas as pl
from jax.experimental.pallas import tpu as pltpu


def _silu(y):
    return y * (1.0 / (1.0 + jnp.exp(-y)))


def _downc_kernel(h, w, x_ref, w1_ref, b1_ref, w2_ref, b2_ref, w3_ref, b3_ref,
                  o_ref, xt_ref, y1_ref):
    c1 = x_ref.shape[1]
    c_ = w1_ref.shape[1]
    c2h = w2_ref.shape[1]
    ho, wo = h // 2, w // 2
    p = ho * wo

    xt = jnp.transpose(x_ref[0], (1, 0))
    nch = c1 // 128
    for c in range(nch):
        xt_ref[c] = xt[:, c * 128:(c + 1) * 128].reshape(h, w, 128)

    y1 = jnp.dot(xt.astype(jnp.bfloat16), w1_ref[...],
                 preferred_element_type=jnp.float32)
    y1 = _silu(y1 + b1_ref[...])

    y1_ref[0:1, :, :] = jnp.zeros((1, w + 1, c_), jnp.float32)
    y1_ref[:, 0:1, :] = jnp.zeros((h + 1, 1, c_), jnp.float32)
    y1_ref[1:h + 1, 1:w + 1, :] = y1.reshape(h, w, c_)

    taps = []
    for kh in range(3):
        for kw in range(3):
            t = y1_ref[pl.ds(kh, ho, 2), pl.ds(kw, wo, 2), :]
            taps.append(t.reshape(p, c_).astype(jnp.bfloat16))
    patches = jnp.concatenate(taps, axis=1)
    y2 = jax.lax.dot_general(w2_ref[...], patches, (((0,), (1,)), ((), ())),
                             preferred_element_type=jnp.float32)
    y2 = _silu(y2 + b2_ref[...])
    o_ref[0, 0:c2h, :] = y2.astype(o_ref.dtype)

    y3 = None
    for c in range(nch):
        p00 = xt_ref[c, pl.ds(0, ho, 2), pl.ds(0, wo, 2), :]
        p01 = xt_ref[c, pl.ds(0, ho, 2), pl.ds(1, wo, 2), :]
        p10 = xt_ref[c, pl.ds(1, ho, 2), pl.ds(0, wo, 2), :]
        p11 = xt_ref[c, pl.ds(1, ho, 2), pl.ds(1, wo, 2), :]
        xm = jnp.maximum(jnp.maximum(p00, p01), jnp.maximum(p10, p11))
        xm = xm.astype(jnp.bfloat16).reshape(p, 128)
        part = jax.lax.dot_general(w3_ref[c * 128:(c + 1) * 128, :], xm,
                                   (((0,), (1,)), ((), ())),
                                   preferred_element_type=jnp.float32)
        y3 = part if y3 is None else y3 + part
    y3 = _silu(y3 + b3_ref[...])
    o_ref[0, c2h:2 * c2h, :] = y3.astype(o_ref.dtype)


def kernel(x, w1, s1, b1, w2, s2, b2, w3, s3, b3):
    n, c1, h, w = x.shape
    c_ = w1.shape[0]
    c2h = w2.shape[0]
    ho, wo = h // 2, w // 2
    p = ho * wo

    x3 = x.reshape(n, c1, h * w)

    w1s = (w1.reshape(c_, c1) * s1[:, None]).T.astype(jnp.bfloat16)
    b1r = b1.reshape(1, c_).astype(jnp.float32)
    w2s = (jnp.transpose(w2, (2, 3, 1, 0)) * s2).reshape(9 * c_, c2h)
    w2s = w2s.astype(jnp.bfloat16)
    b2c = b2.reshape(c2h, 1).astype(jnp.float32)
    w3s = (w3.reshape(c2h, c1) * s3[:, None]).T.astype(jnp.bfloat16)
    b3c = b3.reshape(c2h, 1).astype(jnp.float32)

    import functools
    body = functools.partial(_downc_kernel, h, w)

    out = pl.pallas_call(
        body,
        out_shape=jax.ShapeDtypeStruct((n, 2 * c2h, p), x.dtype),
        grid=(n,),
        in_specs=[
            pl.BlockSpec((1, c1, h * w), lambda i: (i, 0, 0)),
            pl.BlockSpec((c1, c_), lambda i: (0, 0)),
            pl.BlockSpec((1, c_), lambda i: (0, 0)),
            pl.BlockSpec((9 * c_, c2h), lambda i: (0, 0)),
            pl.BlockSpec((c2h, 1), lambda i: (0, 0)),
            pl.BlockSpec((c1, c2h), lambda i: (0, 0)),
            pl.BlockSpec((c2h, 1), lambda i: (0, 0)),
        ],
        out_specs=pl.BlockSpec((1, 2 * c2h, p), lambda i: (i, 0, 0)),
        scratch_shapes=[
            pltpu.VMEM((c1 // 128, h, w, 128), jnp.float32),
            pltpu.VMEM((h + 1, w + 1, c_), jnp.float32),
        ],
        compiler_params=pltpu.CompilerParams(
            dimension_semantics=("parallel",)),
    )(x3, w1s, b1r, w2s, b2c, w3s, b3c)

    return out.reshape(n, 2 * c2h, ho, wo)
